# Initial kernel scaffold; baseline (speedup 1.0000x reference)
import jax
import jax.numpy as jnp
from jax.experimental import pallas as pl


def kernel(x, embed_weight):
    raise NotImplementedError("write your pallas kernel here")


# speedup vs baseline: 1.4818x; 1.4818x over previous
import functools

import jax
import jax.numpy as jnp
from jax import lax
from jax.experimental import pallas as pl
from jax.experimental.pallas import tpu as pltpu
from jax.experimental.pallas import tpu_sc as plsc

B, N, D = 4, 2048, 2048
NUM_CORES = 2
NUM_SUBCORES = 16
NW = NUM_CORES * NUM_SUBCORES
ROWS_PER_W = N // NW
CHUNK = 16
NCHUNK = ROWS_PER_W // CHUNK

_mesh = plsc.VectorSubcoreMesh(core_axis_name="c", subcore_axis_name="s")


@functools.partial(
    pl.kernel,
    mesh=_mesh,
    out_type=jax.ShapeDtypeStruct((B * N, D), jnp.float32),
    scratch_types=[
        pltpu.VMEM((CHUNK, D), jnp.float32),
        pltpu.SemaphoreType.DMA,
    ],
)
def _bcast_copy(w_hbm, out_hbm, buf, sem):
    wid = lax.axis_index("s") * NUM_CORES + lax.axis_index("c")
    base = wid * ROWS_PER_W
    for i in range(NCHUNK):
        r0 = base + i * CHUNK
        pltpu.sync_copy(w_hbm.at[pl.ds(r0, CHUNK), :], buf)
        for b in range(B):
            pltpu.sync_copy(buf, out_hbm.at[pl.ds(b * N + r0, CHUNK), :])


def kernel(x, embed_weight):
    b, n = x.shape
    out = _bcast_copy(embed_weight)
    return out.reshape(b, n, D)


# speedup vs baseline: 1.5503x; 1.0462x over previous
import functools

import jax
import jax.numpy as jnp
from jax import lax
from jax.experimental import pallas as pl
from jax.experimental.pallas import tpu as pltpu
from jax.experimental.pallas import tpu_sc as plsc

B, N, D = 4, 2048, 2048
NUM_CORES = 2
NUM_SUBCORES = 16
NW = NUM_CORES * NUM_SUBCORES
ROWS_PER_W = N // NW
CHUNK = 16
NCHUNK = ROWS_PER_W // CHUNK
NBUF = 3

_mesh = plsc.VectorSubcoreMesh(core_axis_name="c", subcore_axis_name="s")


@functools.partial(
    pl.kernel,
    mesh=_mesh,
    out_type=jax.ShapeDtypeStruct((B * N, D), jnp.float32),
    scratch_types=(
        [pltpu.VMEM((CHUNK, D), jnp.float32) for _ in range(NBUF)]
        + [pltpu.SemaphoreType.DMA for _ in range(2 * NBUF)]
    ),
)
def _bcast_copy(w_hbm, out_hbm, *scratch):
    bufs = scratch[:NBUF]
    rsem = scratch[NBUF:2 * NBUF]
    wsem = scratch[2 * NBUF:]
    wid = lax.axis_index("s") * NUM_CORES + lax.axis_index("c")
    base = wid * ROWS_PER_W

    reads = [None] * NCHUNK
    writes = [None] * NCHUNK
    drained = set()

    def start_read(i):
        r0 = base + i * CHUNK
        reads[i] = pltpu.async_copy(
            w_hbm.at[pl.ds(r0, CHUNK), :], bufs[i % NBUF], rsem[i % NBUF])

    for i in range(min(NBUF, NCHUNK)):
        start_read(i)
    for i in range(NCHUNK):
        reads[i].wait()
        r0 = base + i * CHUNK
        writes[i] = [
            pltpu.async_copy(bufs[i % NBUF],
                             out_hbm.at[pl.ds(b * N + r0, CHUNK), :],
                             wsem[i % NBUF])
            for b in range(B)
        ]
        j = i + NBUF - 1
        if NBUF <= j < NCHUNK:
            for h in writes[j - NBUF]:
                h.wait()
            drained.add(j - NBUF)
            start_read(j)
    for i in range(NCHUNK):
        if i not in drained:
            for h in writes[i]:
                h.wait()


def kernel(x, embed_weight):
    b, n = x.shape
    out = _bcast_copy(embed_weight)
    return out.reshape(b, n, D)
